# baseline (speedup 1.0000x reference)
import functools
import jax
import jax.numpy as jnp
from jax import lax
from jax.experimental import pallas as pl
from jax.experimental.pallas import tpu as pltpu
from jax.experimental.pallas import tpu_sc as plsc

_N_EMBED = 16
_BATCH = 16384
_NC = 2
_NS = 16
_NW = _NC * _NS
_B_PER_W = _BATCH // _NW
_CHUNK = 128
_N_CHUNKS = _B_PER_W // _CHUNK


def _gather_body(table_hbm, idx_hbm, out_hbm, idx_v, rows_v, gsem, osem):
    wid = lax.axis_index("s") * _NC + lax.axis_index("c")
    pltpu.sync_copy(idx_hbm.at[wid], idx_v)
    gathers = [
        pltpu.async_copy(table_hbm.at[idx_v.at[j]], rows_v.at[j], gsem)
        for j in range(_N_CHUNKS)
    ]
    outs = []
    for j in range(_N_CHUNKS):
        gathers[j].wait()
        outs.append(pltpu.async_copy(rows_v.at[j], out_hbm.at[wid, j], osem))
    for cp in outs:
        cp.wait()


@functools.partial(jax.jit, donate_argnums=())
def _gather(table, idx):
    call = pl.kernel(
        _gather_body,
        out_type=jax.ShapeDtypeStruct((_NW, _N_CHUNKS, _CHUNK, _N_EMBED),
                                      jnp.float32),
        mesh=plsc.VectorSubcoreMesh(core_axis_name="c", subcore_axis_name="s"),
        compiler_params=pltpu.CompilerParams(use_tc_tiling_on_sc=False),
        scratch_types=[
            pltpu.VMEM((_N_CHUNKS, _CHUNK), jnp.int32),
            pltpu.VMEM((_N_CHUNKS, _CHUNK, _N_EMBED), jnp.float32),
            pltpu.SemaphoreType.DMA,
            pltpu.SemaphoreType.DMA,
        ],
    )
    return call(table, idx)


def kernel(input_words, in_table):
    idx = input_words.astype(jnp.int32).reshape(_NW, _N_CHUNKS, _CHUNK)
    out = _gather(in_table, idx)
    return out.reshape(_BATCH, _N_EMBED)
